# baseline (speedup 1.0000x reference)
import jax
import jax.numpy as jnp
from jax import lax
from jax.experimental import pallas as pl
from jax.experimental.pallas import tpu as pltpu

_IN = 768
_E = 256


def _fused_body(x_ref, w_emb_ref, b_emb_ref, w_a1_ref, b_a1_ref,
                w_a2_ref, b_a2_ref, w_fc_ref, b_fc_ref, w_out_ref, b_out_ref,
                out_ref):
    c0 = (((0,), (1,)), ((), ()))
    w_tail = lax.dot_general(w_out_ref[...], w_fc_ref[...], c0,
                             preferred_element_type=jnp.float32)
    b_tail = jnp.dot(b_fc_ref[...], w_out_ref[...],
                     preferred_element_type=jnp.float32) + b_out_ref[...]

    tb = x_ref.shape[0]
    half = tb // 2
    contract = (((1,), (1,)), ((), ()))
    for i in range(2):
        sl = pl.ds(i * half, half)
        x = x_ref[sl, :]
        h = jnp.dot(x, w_emb_ref[...],
                    preferred_element_type=jnp.float32) + b_emb_ref[...]
        t = jnp.dot(h, w_a1_ref[...],
                    preferred_element_type=jnp.float32) + b_a1_ref[...]
        t = jnp.maximum(t, 0.0)
        a = jnp.dot(t, w_a2_ref[...],
                    preferred_element_type=jnp.float32) + b_a2_ref[...]
        e = jnp.exp(a)
        num = lax.dot_general(w_tail, e * h, contract,
                              preferred_element_type=jnp.float32)
        den = lax.dot_general(jnp.ones((1, _E), jnp.float32), e, contract,
                              preferred_element_type=jnp.float32)
        out_ref[:, sl] = num * pl.reciprocal(den, approx=True) + b_tail


def kernel(x, w_emb, b_emb, w_a1, b_a1, w_a2, b_a2, w_fc, b_fc, w_out, b_out):
    B, IN = x.shape
    assert IN == _IN

    Bp = ((B + 255) // 256) * 256
    tb = next(t for t in (1024, 512, 256) if Bp % t == 0)
    if Bp != B:
        x = jnp.pad(x, ((0, Bp - B), (0, 0)))

    full = lambda shape: pl.BlockSpec(shape, lambda i: (0, 0))
    out = pl.pallas_call(
        _fused_body,
        out_shape=jax.ShapeDtypeStruct((1, Bp), jnp.float32),
        grid=(Bp // tb,),
        in_specs=[
            pl.BlockSpec((tb, IN), lambda i: (i, 0)),
            full((IN, _E)), full((1, _E)),
            full((_E, _E)), full((1, _E)),
            full((_E, _E)), full((1, _E)),
            full((_E, _E)), full((1, _E)),
            full((_E, 1)), full((1, 1)),
        ],
        out_specs=pl.BlockSpec((1, tb), lambda i: (0, i)),
        compiler_params=pltpu.CompilerParams(
            dimension_semantics=("parallel",)),
    )(x, w_emb, b_emb, w_a1, b_a1, w_a2, b_a2, w_fc, b_fc, w_out, b_out)

    return out.reshape(Bp, 1)[:B]


# speedup vs baseline: 1.1314x; 1.1314x over previous
import jax
import jax.numpy as jnp
from jax import lax
from jax.experimental import pallas as pl
from jax.experimental.pallas import tpu as pltpu

_IN = 768
_E = 256


def _fused_body(x_ref, w_emb_ref, b_emb_ref, w_a1_ref, b_a1_ref,
                w_a2_ref, b_a2_ref, w_fc_ref, b_fc_ref, w_out_ref, b_out_ref,
                out_ref):
    c0 = (((0,), (1,)), ((), ()))
    w_tail = lax.dot_general(w_out_ref[...], w_fc_ref[...], c0,
                             preferred_element_type=jnp.float32)
    b_tail = jnp.dot(b_fc_ref[...], w_out_ref[...],
                     preferred_element_type=jnp.float32) + b_out_ref[...]

    tb = x_ref.shape[0]
    half = tb // 2
    contract = (((1,), (1,)), ((), ()))
    for i in range(2):
        sl = pl.ds(i * half, half)
        x = x_ref[sl, :]
        h = jnp.dot(x, w_emb_ref[...],
                    preferred_element_type=jnp.float32) + b_emb_ref[...]
        t = jnp.dot(h, w_a1_ref[...],
                    preferred_element_type=jnp.float32) + b_a1_ref[...]
        t = jnp.maximum(t, 0.0)
        a = jnp.dot(t, w_a2_ref[...],
                    preferred_element_type=jnp.float32) + b_a2_ref[...]
        e = jnp.exp(a)
        num = lax.dot_general(w_tail, e * h, contract,
                              preferred_element_type=jnp.float32)
        den = lax.dot_general(jnp.ones((1, _E), jnp.float32), e, contract,
                              preferred_element_type=jnp.float32)
        out_ref[:, sl] = num * pl.reciprocal(den, approx=True) + b_tail


def kernel(x, w_emb, b_emb, w_a1, b_a1, w_a2, b_a2, w_fc, b_fc, w_out, b_out):
    B, IN = x.shape
    assert IN == _IN

    Bp = ((B + 255) // 256) * 256
    tb = next(t for t in (4096, 2048, 1024, 512, 256) if Bp % t == 0)
    if Bp != B:
        x = jnp.pad(x, ((0, Bp - B), (0, 0)))

    full = lambda shape: pl.BlockSpec(shape, lambda i: (0, 0))
    out = pl.pallas_call(
        _fused_body,
        out_shape=jax.ShapeDtypeStruct((1, Bp), jnp.float32),
        grid=(Bp // tb,),
        in_specs=[
            pl.BlockSpec((tb, IN), lambda i: (i, 0)),
            full((IN, _E)), full((1, _E)),
            full((_E, _E)), full((1, _E)),
            full((_E, _E)), full((1, _E)),
            full((_E, _E)), full((1, _E)),
            full((_E, 1)), full((1, 1)),
        ],
        out_specs=pl.BlockSpec((1, tb), lambda i: (0, i)),
        compiler_params=pltpu.CompilerParams(
            dimension_semantics=("parallel",)),
    )(x, w_emb, b_emb, w_a1, b_a1, w_a2, b_a2, w_fc, b_fc, w_out, b_out)

    return out.reshape(Bp, 1)[:B]


# speedup vs baseline: 1.1357x; 1.0038x over previous
import jax
import jax.numpy as jnp
from jax import lax
from jax.experimental import pallas as pl
from jax.experimental.pallas import tpu as pltpu

_IN = 768
_E = 256


def _fused_body(x_ref, w_emb_ref, b_emb_ref, w_a1_ref, b_a1_ref,
                w_a2_ref, b_a2_ref, w_fc_ref, b_fc_ref, w_out_ref, b_out_ref,
                out_ref):
    c0 = (((0,), (1,)), ((), ()))
    w_tail = lax.dot_general(w_out_ref[...], w_fc_ref[...], c0,
                             preferred_element_type=jnp.float32)
    b_tail = jnp.dot(b_fc_ref[...], w_out_ref[...],
                     preferred_element_type=jnp.float32) + b_out_ref[...]

    tb = x_ref.shape[0]
    half = tb // 2
    contract = (((1,), (1,)), ((), ()))
    for i in range(2):
        sl = pl.ds(i * half, half)
        x = x_ref[sl, :].astype(jnp.bfloat16)
        h = jnp.dot(x, w_emb_ref[...].astype(jnp.bfloat16),
                    preferred_element_type=jnp.float32) + b_emb_ref[...]
        t = jnp.dot(h.astype(jnp.bfloat16), w_a1_ref[...].astype(jnp.bfloat16),
                    preferred_element_type=jnp.float32) + b_a1_ref[...]
        t = jnp.maximum(t, 0.0)
        a = jnp.dot(t.astype(jnp.bfloat16), w_a2_ref[...].astype(jnp.bfloat16),
                    preferred_element_type=jnp.float32) + b_a2_ref[...]
        e = jnp.exp(a)
        num = lax.dot_general(w_tail, e * h, contract,
                              preferred_element_type=jnp.float32)
        den = lax.dot_general(jnp.ones((1, _E), jnp.float32), e, contract,
                              preferred_element_type=jnp.float32)
        out_ref[:, sl] = num * pl.reciprocal(den, approx=True) + b_tail


def kernel(x, w_emb, b_emb, w_a1, b_a1, w_a2, b_a2, w_fc, b_fc, w_out, b_out):
    B, IN = x.shape
    assert IN == _IN

    Bp = ((B + 255) // 256) * 256
    tb = next(t for t in (2048, 1024, 512, 256) if Bp % t == 0)
    if Bp != B:
        x = jnp.pad(x, ((0, Bp - B), (0, 0)))

    full = lambda shape: pl.BlockSpec(shape, lambda i: (0, 0))
    out = pl.pallas_call(
        _fused_body,
        out_shape=jax.ShapeDtypeStruct((1, Bp), jnp.float32),
        grid=(Bp // tb,),
        in_specs=[
            pl.BlockSpec((tb, IN), lambda i: (i, 0)),
            full((IN, _E)), full((1, _E)),
            full((_E, _E)), full((1, _E)),
            full((_E, _E)), full((1, _E)),
            full((_E, _E)), full((1, _E)),
            full((_E, 1)), full((1, 1)),
        ],
        out_specs=pl.BlockSpec((1, tb), lambda i: (0, i)),
        compiler_params=pltpu.CompilerParams(
            dimension_semantics=("parallel",)),
    )(x, w_emb, b_emb, w_a1, b_a1, w_a2, b_a2, w_fc, b_fc, w_out, b_out)

    return out.reshape(Bp, 1)[:B]
